# baseline (speedup 1.0000x reference)
import functools

import jax
import jax.numpy as jnp
from jax.experimental import pallas as pl
from jax.experimental.pallas import tpu as pltpu

_EPS = 1e-5
_ROWS = 400


def _bn_relu(y, g, b):
    return jnp.maximum(g * (y * (1.0 / jnp.sqrt(1.0 + _EPS))) + b, 0.0)


def _fused_body(dims, adj_ref, x_ref, p_ref, out_ref, s2_ref):
    nblk, nfeat, nhid, nmid, nclass = dims
    i = pl.program_id(0)

    w1 = p_ref[0:nfeat, 0:nhid]
    w2 = p_ref[nfeat:nfeat + nhid, 0:nhid]
    r0 = nfeat + nhid
    m1w = p_ref[r0:r0 + nhid, 0:nmid]
    r1 = r0 + nhid
    m2w = p_ref[r1:r1 + nmid, 0:nhid]
    r2 = r1 + nmid
    m3w = p_ref[r2:r2 + nhid, 0:nclass]
    r3 = r2 + nhid

    def vec(k, width):
        return p_ref[r3 + k:r3 + k + 1, 0:width]

    b1, g1, be1, b2 = vec(0, nhid), vec(1, nhid), vec(2, nhid), vec(3, nhid)
    m1b, m1g, m1be = vec(4, nmid), vec(5, nmid), vec(6, nmid)
    m2b, m2g, m2be = vec(7, nhid), vec(8, nhid), vec(9, nhid)
    m3b = vec(10, nclass)

    @pl.when(i < nblk)
    def _pass1():
        s1 = jnp.dot(x_ref[...], w1, preferred_element_type=jnp.float32)
        y = jnp.dot(adj_ref[...], s1, preferred_element_type=jnp.float32)
        x1 = _bn_relu(y + b1, g1, be1)
        s2_ref[pl.ds(i * _ROWS, _ROWS), :] = jnp.dot(
            x1, w2, preferred_element_type=jnp.float32)

    @pl.when(i >= nblk)
    def _pass2():
        y = jnp.dot(adj_ref[...], s2_ref[...],
                    preferred_element_type=jnp.float32)
        x2 = _bn_relu(y + b2, g1, be1)
        h = _bn_relu(jnp.dot(x2, m1w, preferred_element_type=jnp.float32)
                     + m1b, m1g, m1be)
        h = _bn_relu(jnp.dot(h, m2w, preferred_element_type=jnp.float32)
                     + m2b, m2g, m2be)
        o = jnp.dot(h, m3w, preferred_element_type=jnp.float32) + m3b
        m = jnp.max(o, axis=1, keepdims=True)
        lse = jnp.log(jnp.sum(jnp.exp(o - m), axis=1, keepdims=True)) + m
        out_ref[...] = o - lse


def _const_spec(shape):
    return pl.BlockSpec(shape, lambda i: (0,) * len(shape))


def kernel(x, adj, W1, b1, W2, b2, bn1_g, bn1_b, m1_W, m1_b, m1_g, m1_be,
           m2_W, m2_b, m2_g, m2_be, m3_W, m3_b):
    n, nfeat = x.shape
    nhid = W1.shape[1]
    nmid = m1_W.shape[1]
    nclass = m3_W.shape[1]
    f32 = jnp.float32
    nblk = n // _ROWS

    lane = max(nfeat, nmid)

    def padc(a):
        return jnp.pad(a, ((0, 0), (0, lane - a.shape[1])))

    vecs = [b1, bn1_g, bn1_b, b2, m1_b, m1_g, m1_be, m2_b, m2_g, m2_be, m3_b]
    packed = jnp.concatenate(
        [padc(W1), padc(W2), padc(m1_W), padc(m2_W), padc(m3_W)]
        + [padc(v.reshape(1, -1)) for v in vecs], axis=0)
    nrows = packed.shape[0]
    pad_rows = (-nrows) % 8
    packed = jnp.pad(packed, ((0, pad_rows), (0, 0)))

    def adj_map(i):
        return (jnp.where(i < nblk, i, i - nblk), 0)

    def out_map(i):
        return (jnp.maximum(i - nblk, 0), 0)

    body = functools.partial(_fused_body, (nblk, nfeat, nhid, nmid, nclass))

    out = pl.pallas_call(
        body,
        grid=(2 * nblk,),
        in_specs=[pl.BlockSpec((_ROWS, n), adj_map),
                  _const_spec((n, nfeat)),
                  _const_spec(packed.shape)],
        out_specs=pl.BlockSpec((_ROWS, nclass), out_map),
        out_shape=jax.ShapeDtypeStruct((n, nclass), f32),
        scratch_shapes=[pltpu.VMEM((n, nhid), f32)],
        compiler_params=pltpu.CompilerParams(
            dimension_semantics=("arbitrary",)),
    )(adj, x, packed)
    return out


# speedup vs baseline: 1.0289x; 1.0289x over previous
import functools

import jax
import jax.numpy as jnp
from jax.experimental import pallas as pl
from jax.experimental.pallas import tpu as pltpu

_EPS = 1e-5
_ROWS = 400


def _bn_relu(y, g, b):
    return jnp.maximum(g * (y * (1.0 / jnp.sqrt(1.0 + _EPS))) + b, 0.0)


def _fused_body(nblk, adj_ref, x_ref, w1_ref, b1_ref, g_ref, be_ref,
                w2_ref, b2_ref, m1w_ref, m1b_ref, m1g_ref, m1be_ref,
                m2w_ref, m2b_ref, m2g_ref, m2be_ref, m3w_ref, m3b_ref,
                out_ref, s2_ref):
    i = pl.program_id(0)

    @pl.when(i < nblk)
    def _pass1():
        s1 = jnp.dot(x_ref[...], w1_ref[...],
                     preferred_element_type=jnp.float32)
        y = jnp.dot(adj_ref[...], s1, preferred_element_type=jnp.float32)
        x1 = _bn_relu(y + b1_ref[...], g_ref[...], be_ref[...])
        s2_ref[pl.ds(i * _ROWS, _ROWS), :] = jnp.dot(
            x1, w2_ref[...], preferred_element_type=jnp.float32)

    @pl.when(i >= nblk)
    def _pass2():
        y = jnp.dot(adj_ref[...], s2_ref[...],
                    preferred_element_type=jnp.float32)
        x2 = _bn_relu(y + b2_ref[...], g_ref[...], be_ref[...])
        h = _bn_relu(jnp.dot(x2, m1w_ref[...],
                             preferred_element_type=jnp.float32)
                     + m1b_ref[...], m1g_ref[...], m1be_ref[...])
        h = _bn_relu(jnp.dot(h, m2w_ref[...],
                             preferred_element_type=jnp.float32)
                     + m2b_ref[...], m2g_ref[...], m2be_ref[...])
        o = jnp.dot(h, m3w_ref[...],
                    preferred_element_type=jnp.float32) + m3b_ref[...]
        m = jnp.max(o, axis=1, keepdims=True)
        lse = jnp.log(jnp.sum(jnp.exp(o - m), axis=1, keepdims=True)) + m
        out_ref[...] = o - lse


def _const_spec(shape):
    return pl.BlockSpec(shape, lambda i: (0,) * len(shape))


def kernel(x, adj, W1, b1, W2, b2, bn1_g, bn1_b, m1_W, m1_b, m1_g, m1_be,
           m2_W, m2_b, m2_g, m2_be, m3_W, m3_b):
    n, nfeat = x.shape
    nhid = W1.shape[1]
    nmid = m1_W.shape[1]
    nclass = m3_W.shape[1]
    f32 = jnp.float32
    nblk = n // _ROWS

    def row(v):
        return v.reshape(1, -1)

    def adj_map(i):
        return (jnp.where(i < nblk, i, i - nblk), 0)

    def out_map(i):
        return (jnp.maximum(i - nblk, 0), 0)

    body = functools.partial(_fused_body, nblk)

    out = pl.pallas_call(
        body,
        grid=(2 * nblk,),
        in_specs=[pl.BlockSpec((_ROWS, n), adj_map),
                  _const_spec((n, nfeat)),
                  _const_spec((nfeat, nhid)), _const_spec((1, nhid)),
                  _const_spec((1, nhid)), _const_spec((1, nhid)),
                  _const_spec((nhid, nhid)), _const_spec((1, nhid)),
                  _const_spec((nhid, nmid)), _const_spec((1, nmid)),
                  _const_spec((1, nmid)), _const_spec((1, nmid)),
                  _const_spec((nmid, nhid)), _const_spec((1, nhid)),
                  _const_spec((1, nhid)), _const_spec((1, nhid)),
                  _const_spec((nhid, nclass)), _const_spec((1, nclass))],
        out_specs=pl.BlockSpec((_ROWS, nclass), out_map),
        out_shape=jax.ShapeDtypeStruct((n, nclass), f32),
        scratch_shapes=[pltpu.VMEM((n, nhid), f32)],
        compiler_params=pltpu.CompilerParams(
            dimension_semantics=("arbitrary",)),
    )(adj, x, W1, row(b1), row(bn1_g), row(bn1_b), W2, row(b2),
      m1_W, row(m1_b), row(m1_g), row(m1_be),
      m2_W, row(m2_b), row(m2_g), row(m2_be),
      m3_W, row(m3_b))
    return out
